# baseline (speedup 1.0000x reference)
import functools

import jax
import jax.numpy as jnp
from jax import lax
from jax.experimental import pallas as pl
from jax.experimental.pallas import tpu as pltpu
from jax.experimental.pallas import tpu_sc as plsc

N = 10000
E = 320000
D = 128

CHUNK = 128
NC = 2
NS = 16
NW = NC * NS
CPW = 80
EPAD = NW * CPW * CHUNK
NBUF = 2
PHASE = 40
INNER = PHASE // NBUF

ACC_ROWS = N + 8
ROWS_PER_TILE = 624
REM0 = NS * ROWS_PER_TILE


def _sc_aggregate(x, src2d, dst2d, zeros):
    mesh = plsc.VectorSubcoreMesh(core_axis_name="c", subcore_axis_name="s")

    @functools.partial(
        pl.kernel,
        mesh=mesh,
        out_type=jax.ShapeDtypeStruct((NC, N, D), jnp.float32),
        scratch_types=[
            pltpu.VMEM((PHASE, CHUNK), jnp.int32),
            pltpu.VMEM((PHASE, CHUNK), jnp.int32),
            pltpu.VMEM((CHUNK, D), jnp.float32),
            pltpu.VMEM((CHUNK, D), jnp.float32),
            pltpu.VMEM_SHARED((ACC_ROWS, D), jnp.float32),
            pltpu.SemaphoreType.DMA,
            pltpu.SemaphoreType.DMA,
            pltpu.SemaphoreType.DMA,
            pltpu.SemaphoreType.DMA,
        ],
    )
    def agg_kernel(x_hbm, src_hbm, dst_hbm, zero_hbm, out_hbm,
                   src_v, dst_v, r0, r1, acc, g0, g1, s0, s1):
        rows = (r0, r1)
        gs = (g0, g1)
        ss = (s0, s1)
        c = lax.axis_index("c")
        sid = lax.axis_index("s")
        w = c * NS + sid
        row0 = sid * ROWS_PER_TILE

        pltpu.sync_copy(zero_hbm.at[pl.ds(row0, ROWS_PER_TILE)],
                        acc.at[pl.ds(row0, ROWS_PER_TILE)])

        @pl.when(sid == 0)
        def _():
            pltpu.sync_copy(zero_hbm.at[pl.ds(REM0, ACC_ROWS - REM0)],
                            acc.at[pl.ds(REM0, ACC_ROWS - REM0)])

        plsc.subcore_barrier()

        for p in range(CPW // PHASE):
            base = w * CPW + p * PHASE
            pltpu.sync_copy(src_hbm.at[pl.ds(base, PHASE)], src_v)
            pltpu.sync_copy(dst_hbm.at[pl.ds(base, PHASE)], dst_v)

            for b in range(NBUF):
                pltpu.async_copy(x_hbm.at[src_v.at[b]], rows[b], gs[b])

            def inner(t, carry):
                for b in range(NBUF):
                    j = t * NBUF + b
                    pltpu.make_async_copy(
                        x_hbm.at[src_v.at[j]], rows[b], gs[b]).wait()
                    pltpu.async_copy(
                        rows[b], acc.at[dst_v.at[j]], ss[b], add=True)
                    pltpu.make_async_copy(
                        rows[b], acc.at[dst_v.at[j]], ss[b]).wait()

                    @pl.when(t < INNER - 1)
                    def _():
                        pltpu.async_copy(
                            x_hbm.at[src_v.at[j + NBUF]], rows[b], gs[b])
                return carry

            lax.fori_loop(0, INNER, inner, 0)

        plsc.subcore_barrier()

        pltpu.sync_copy(acc.at[pl.ds(row0, ROWS_PER_TILE)],
                        out_hbm.at[c, pl.ds(row0, ROWS_PER_TILE)])

        @pl.when(sid == 0)
        def _():
            pltpu.sync_copy(acc.at[pl.ds(REM0, N - REM0)],
                            out_hbm.at[c, pl.ds(REM0, N - REM0)])

    return agg_kernel(x, src2d, dst2d, zeros)


BLK = 1000


def _mlp_body(eps_ref, x_ref, p_ref, w1_ref, b1_ref, w2_ref, b2_ref, o_ref):
    agg = p_ref[0] + p_ref[1]
    out = (1.0 + eps_ref[...]) * x_ref[...] + agg
    h = jnp.dot(out, w1_ref[...], preferred_element_type=jnp.float32)
    h = jnp.maximum(h + b1_ref[...], 0.0)
    o_ref[...] = (
        jnp.dot(h, w2_ref[...], preferred_element_type=jnp.float32)
        + b2_ref[...]
    )


def _mlp(x, partials, eps, W1, b1, W2, b2):
    eps2 = eps.reshape(1, 1).astype(jnp.float32)
    return pl.pallas_call(
        _mlp_body,
        grid=(N // BLK,),
        in_specs=[
            pl.BlockSpec((1, 1), lambda i: (0, 0)),
            pl.BlockSpec((BLK, D), lambda i: (i, 0)),
            pl.BlockSpec((NC, BLK, D), lambda i: (0, i, 0)),
            pl.BlockSpec((D, D), lambda i: (0, 0)),
            pl.BlockSpec((1, D), lambda i: (0, 0)),
            pl.BlockSpec((D, D), lambda i: (0, 0)),
            pl.BlockSpec((1, D), lambda i: (0, 0)),
        ],
        out_specs=pl.BlockSpec((BLK, D), lambda i: (i, 0)),
        out_shape=jax.ShapeDtypeStruct((N, D), jnp.float32),
    )(eps2, x, partials, W1, b1.reshape(1, D), W2, b2.reshape(1, D))


@jax.jit
def kernel(x, edge_idx, eps, W1, b1, W2, b2):
    ei = edge_idx.astype(jnp.int32)
    src = jnp.concatenate(
        [ei[0], jnp.zeros((EPAD - E,), jnp.int32)]).reshape(-1, CHUNK)
    dst = jnp.concatenate(
        [ei[1], jnp.full((EPAD - E,), N, jnp.int32)]).reshape(-1, CHUNK)
    zeros = jnp.zeros((ACC_ROWS, D), jnp.float32)
    partials = _sc_aggregate(x, src, dst, zeros)
    return _mlp(x, partials, eps, W1, b1, W2, b2)
